# Initial kernel scaffold; baseline (speedup 1.0000x reference)
import jax
import jax.numpy as jnp
from jax.experimental import pallas as pl


def kernel(x, targets):
    raise NotImplementedError("write your pallas kernel here")


# speedup vs baseline: 1.0019x; 1.0019x over previous
import jax
import jax.numpy as jnp
from jax.experimental import pallas as pl

_ANCHOR_W = (116.0, 156.0, 373.0)
_ANCHOR_H = (90.0, 198.0, 326.0)
_G = 64
_STRIDE = 16.0
_NC = 11


def _decode_body(x_ref, o_ref):
    a = pl.program_id(1)
    v = x_ref[0]

    i = jax.lax.broadcasted_iota(jnp.int32, (1, _G * _G), 1)
    gx = (i % _G).astype(jnp.float32)
    gy = (i // _G).astype(jnp.float32)

    aw = jnp.where(a == 0, _ANCHOR_W[0], jnp.where(a == 1, _ANCHOR_W[1], _ANCHOR_W[2]))
    ah = jnp.where(a == 0, _ANCHOR_H[0], jnp.where(a == 1, _ANCHOR_H[1], _ANCHOR_H[2]))

    bx = (jax.nn.sigmoid(v[0:1]) + gx) * _STRIDE
    by = (jax.nn.sigmoid(v[1:2]) + gy) * _STRIDE
    bw = jnp.exp(v[2:3]) * aw
    bh = jnp.exp(v[3:4]) * ah
    conf = jax.nn.sigmoid(v[4:5])

    logits = v[5:5 + _NC]
    m = jnp.max(logits, axis=0, keepdims=True)
    e = jnp.exp(logits - m)
    s = jnp.sum(e, axis=0, keepdims=True)
    cls = e / s

    res = jnp.concatenate([bx, by, bw, bh, conf, cls], axis=0)
    o_ref[0] = res.T


def kernel(x, targets):
    bs = x.shape[0]
    n_anchor = 3
    attrs = 5 + _NC
    xr = x.reshape(bs, n_anchor * attrs, _G * _G)
    return pl.pallas_call(
        _decode_body,
        grid=(bs, n_anchor),
        in_specs=[pl.BlockSpec((1, attrs, _G * _G), lambda b, a: (b, a, 0))],
        out_specs=pl.BlockSpec((1, _G * _G, attrs), lambda b, a: (b, a, 0)),
        out_shape=jax.ShapeDtypeStruct((bs, n_anchor * _G * _G, attrs), jnp.float32),
    )(xr)


# speedup vs baseline: 1.0030x; 1.0011x over previous
import jax
import jax.numpy as jnp
from jax.experimental import pallas as pl
from jax.experimental.pallas import tpu as pltpu

_ANCHOR_W = (116.0, 156.0, 373.0)
_ANCHOR_H = (90.0, 198.0, 326.0)
_G = 64
_STRIDE = 16.0
_NC = 11


def _decode_body(x_ref, o_ref):
    a = pl.program_id(1)
    v = x_ref[0]

    i = jax.lax.broadcasted_iota(jnp.int32, (1, _G * _G), 1)
    gx = (i % _G).astype(jnp.float32)
    gy = (i // _G).astype(jnp.float32)

    aw = jnp.where(a == 0, _ANCHOR_W[0], jnp.where(a == 1, _ANCHOR_W[1], _ANCHOR_W[2]))
    ah = jnp.where(a == 0, _ANCHOR_H[0], jnp.where(a == 1, _ANCHOR_H[1], _ANCHOR_H[2]))

    bx = (jax.nn.sigmoid(v[0:1]) + gx) * _STRIDE
    by = (jax.nn.sigmoid(v[1:2]) + gy) * _STRIDE
    bw = jnp.exp(v[2:3]) * aw
    bh = jnp.exp(v[3:4]) * ah
    conf = jax.nn.sigmoid(v[4:5])

    logits = v[5:5 + _NC]
    m = jnp.max(logits, axis=0, keepdims=True)
    e = jnp.exp(logits - m)
    s = jnp.sum(e, axis=0, keepdims=True)
    cls = e / s

    res = jnp.concatenate([bx, by, bw, bh, conf, cls], axis=0)
    o_ref[0] = res.T


def kernel(x, targets):
    bs = x.shape[0]
    n_anchor = 3
    attrs = 5 + _NC
    xr = x.reshape(bs, n_anchor * attrs, _G * _G)
    return pl.pallas_call(
        _decode_body,
        grid=(bs, n_anchor),
        in_specs=[pl.BlockSpec((1, attrs, _G * _G), lambda b, a: (b, a, 0))],
        out_specs=pl.BlockSpec((1, _G * _G, attrs), lambda b, a: (b, a, 0)),
        out_shape=jax.ShapeDtypeStruct((bs, n_anchor * _G * _G, attrs), jnp.float32),
        compiler_params=pltpu.CompilerParams(
            dimension_semantics=("parallel", "parallel")
        ),
    )(xr)


# speedup vs baseline: 1.2083x; 1.2047x over previous
import jax
import jax.numpy as jnp
from jax.experimental import pallas as pl
from jax.experimental.pallas import tpu as pltpu

_ANCHOR_W = (116.0, 156.0, 373.0)
_ANCHOR_H = (90.0, 198.0, 326.0)
_G = 64
_STRIDE = 16.0
_NC = 11
_ATTRS = 5 + _NC


def _decode_body(x_ref, o_ref):
    v = x_ref[0]

    i = jax.lax.broadcasted_iota(jnp.int32, (1, _G * _G), 1)
    gx = (i % _G).astype(jnp.float32) * _STRIDE
    gy = (i // _G).astype(jnp.float32) * _STRIDE

    for a in range(3):
        s = v[_ATTRS * a:_ATTRS * (a + 1)]
        bx = jax.nn.sigmoid(s[0:1]) * _STRIDE + gx
        by = jax.nn.sigmoid(s[1:2]) * _STRIDE + gy
        bw = jnp.exp(s[2:3]) * _ANCHOR_W[a]
        bh = jnp.exp(s[3:4]) * _ANCHOR_H[a]
        conf = jax.nn.sigmoid(s[4:5])

        logits = s[5:5 + _NC]
        m = jnp.max(logits, axis=0, keepdims=True)
        e = jnp.exp(logits - m)
        z = jnp.sum(e, axis=0, keepdims=True)
        cls = e / z

        res = jnp.concatenate([bx, by, bw, bh, conf, cls], axis=0)
        o_ref[0, a] = res.T


def kernel(x, targets):
    bs = x.shape[0]
    xr = x.reshape(bs, 3 * _ATTRS, _G * _G)
    out = pl.pallas_call(
        _decode_body,
        grid=(bs,),
        in_specs=[pl.BlockSpec((1, 3 * _ATTRS, _G * _G), lambda b: (b, 0, 0))],
        out_specs=pl.BlockSpec((1, 3, _G * _G, _ATTRS), lambda b: (b, 0, 0, 0)),
        out_shape=jax.ShapeDtypeStruct((bs, 3, _G * _G, _ATTRS), jnp.float32),
        compiler_params=pltpu.CompilerParams(
            dimension_semantics=("parallel",)
        ),
    )(xr)
    return out.reshape(bs, 3 * _G * _G, _ATTRS)
